# baseline (speedup 1.0000x reference)
import functools

import jax
import jax.numpy as jnp
from jax import lax
from jax.experimental import pallas as pl
from jax.experimental.pallas import tpu as pltpu
from jax.experimental.pallas import tpu_sc as plsc

B, G, C, H = 16, 256, 128, 256
TAU, SINKHORN_ITERS = 0.1, 10
NCHUNK = G // 16


def _lse_last(x):
    m = jnp.max(x, axis=-1, keepdims=True)
    return m + jnp.log(jnp.sum(jnp.exp(x - m), axis=-1, keepdims=True))


def _lse_sub(x):
    m = jnp.max(x, axis=-2, keepdims=True)
    return m + jnp.log(jnp.sum(jnp.exp(x - m), axis=-2, keepdims=True))


def _tc_scores_body(gf_ref, w1_ref, b1_ref, w2_ref, b2_ref, p_ref):
    gf = gf_ref[0]
    h = jnp.maximum(
        jnp.dot(gf, w1_ref[...], preferred_element_type=jnp.float32)
        + b1_ref[...], 0.0)
    scores = jnp.dot(h, w2_ref[...], preferred_element_type=jnp.float32) \
        + b2_ref[...]

    la = scores / TAU

    def sk(_, la):
        la = la - _lse_last(la)
        la = la - _lse_sub(la)
        return la

    la = lax.fori_loop(0, SINKHORN_ITERS, sk, la)
    p_ref[0] = jnp.exp(la)


def _store1(ref, idx, val, dtype):
    lanes = lax.iota(jnp.int32, 16)
    plsc.store_scatter(
        ref,
        [jnp.full((16,), idx, jnp.int32)],
        jnp.full((16,), val, dtype),
        mask=lanes == 0,
    )


def _sc_greedy_body(p_hbm, perm_hbm, pv, bound, colmask, supermax, permv):
    cid = lax.axis_index("c")
    sid = lax.axis_index("s")
    wid = sid * 2 + cid

    @pl.when(wid < B)
    def _():
        pltpu.sync_copy(p_hbm.at[wid], pv)
        lanes = lax.iota(jnp.int32, 16)

        def init_cm(k, carry):
            colmask[pl.ds(k * 16, 16)] = jnp.zeros((16,), jnp.float32)
            return carry

        lax.fori_loop(0, NCHUNK, init_cm, 0)

        def init_row(r, carry):
            m = pv[pl.ds(r * G, 16)]
            for k in range(1, NCHUNK):
                m = jnp.maximum(m, pv[pl.ds(r * G + k * 16, 16)])
            _store1(bound, r, jnp.max(m), jnp.float32)
            return carry

        lax.fori_loop(0, G, init_row, 0)

        def init_sm(j, carry):
            _store1(supermax, j, jnp.max(bound[pl.ds(j * 16, 16)]),
                    jnp.float32)
            return carry

        lax.fori_loop(0, 16, init_sm, 0)

        def cond(carry):
            n, it = carry
            return (n < G) & (it < G * G + G)

        def body(carry):
            n, it = carry
            sv = supermax[pl.ds(0, 16)]
            gm = jnp.max(sv)
            j = jnp.min(jnp.where(sv == gm, lanes, 16))
            bv = bound[pl.ds(j * 16, 16)]
            bm = jnp.max(bv)
            r = j * 16 + jnp.min(jnp.where(bv == bm, lanes, 16))
            base = r * G
            best_v = pv[pl.ds(base, 16)] + colmask[pl.ds(0, 16)]
            best_i = lanes
            for k in range(1, NCHUNK):
                v = pv[pl.ds(base + k * 16, 16)] + colmask[pl.ds(k * 16, 16)]
                upd = v > best_v
                best_v = jnp.where(upd, v, best_v)
                best_i = jnp.where(upd, lanes + k * 16, best_i)
            m2 = jnp.max(best_v)
            cnd = jnp.min(jnp.where(best_v == m2, best_i, G))
            eq = m2 >= bm

            @pl.when(eq)
            def _():
                _store1(permv, r, cnd, jnp.int32)
                _store1(colmask, cnd, jnp.float32(-4.0), jnp.float32)
                _store1(bound, r, jnp.float32(-1e9), jnp.float32)

            @pl.when(jnp.logical_not(eq))
            def _():
                _store1(bound, r, m2, jnp.float32)

            _store1(supermax, j, jnp.max(bound[pl.ds(j * 16, 16)]),
                    jnp.float32)
            return (n + jnp.where(eq, 1, 0).astype(jnp.int32), it + 1)

        lax.while_loop(cond, body, (jnp.int32(0), jnp.int32(0)))
        pltpu.sync_copy(permv, perm_hbm.at[wid])


_sc_greedy = functools.partial(
    pl.kernel,
    mesh=plsc.VectorSubcoreMesh(core_axis_name="c", subcore_axis_name="s"),
    out_type=jax.ShapeDtypeStruct((B, G), jnp.int32),
    compiler_params=pltpu.CompilerParams(needs_layout_passes=False),
    scratch_types=[
        pltpu.VMEM((G * G,), jnp.float32),
        pltpu.VMEM((G,), jnp.float32),
        pltpu.VMEM((G,), jnp.float32),
        pltpu.VMEM((16,), jnp.float32),
        pltpu.VMEM((G,), jnp.int32),
    ],
)(_sc_greedy_body)


def _tc_apply_body(perm_ref, cc_ref, gf_ref, rc_ref, rf_ref):
    pcol = perm_ref[0]
    col_ids = lax.broadcasted_iota(jnp.int32, (G, G), 1)
    M = (pcol == col_ids).astype(jnp.float32)
    contract = (((0,), (0,)), ((), ()))
    rf_ref[0] = lax.dot_general(M, gf_ref[0], contract,
                                preferred_element_type=jnp.float32)
    rc_ref[0] = lax.dot_general(M, cc_ref[0], contract,
                                preferred_element_type=jnp.float32)


def kernel(center_coords, group_features, W1, b1, W2, b2):
    b1r = b1.reshape(1, H)
    b2r = b2.reshape(1, G)
    P = pl.pallas_call(
        _tc_scores_body,
        grid=(B,),
        in_specs=[
            pl.BlockSpec((1, G, C), lambda b: (b, 0, 0)),
            pl.BlockSpec((C, H), lambda b: (0, 0)),
            pl.BlockSpec((1, H), lambda b: (0, 0)),
            pl.BlockSpec((H, G), lambda b: (0, 0)),
            pl.BlockSpec((1, G), lambda b: (0, 0)),
        ],
        out_specs=pl.BlockSpec((1, G, G), lambda b: (b, 0, 0)),
        out_shape=jax.ShapeDtypeStruct((B, G, G), jnp.float32),
    )(group_features, W1, b1r, W2, b2r)

    perm = _sc_greedy(P.reshape(B, G * G))

    rc, rf = pl.pallas_call(
        _tc_apply_body,
        grid=(B,),
        in_specs=[
            pl.BlockSpec((1, G, 1), lambda b: (b, 0, 0)),
            pl.BlockSpec((1, G, 3), lambda b: (b, 0, 0)),
            pl.BlockSpec((1, G, C), lambda b: (b, 0, 0)),
        ],
        out_specs=[
            pl.BlockSpec((1, G, 3), lambda b: (b, 0, 0)),
            pl.BlockSpec((1, G, C), lambda b: (b, 0, 0)),
        ],
        out_shape=[
            jax.ShapeDtypeStruct((B, G, 3), jnp.float32),
            jax.ShapeDtypeStruct((B, G, C), jnp.float32),
        ],
    )(perm.reshape(B, G, 1), center_coords, group_features)
    return (rc, rf, perm)


# speedup vs baseline: 1.0480x; 1.0480x over previous
import functools

import jax
import jax.numpy as jnp
from jax import lax
from jax.experimental import pallas as pl
from jax.experimental.pallas import tpu as pltpu
from jax.experimental.pallas import tpu_sc as plsc

B, G, C, H = 16, 256, 128, 256
TAU, SINKHORN_ITERS = 0.1, 10
NCHUNK = G // 16


def _lse_last(x):
    m = jnp.max(x, axis=-1, keepdims=True)
    return m + jnp.log(jnp.sum(jnp.exp(x - m), axis=-1, keepdims=True))


def _lse_sub(x):
    m = jnp.max(x, axis=-2, keepdims=True)
    return m + jnp.log(jnp.sum(jnp.exp(x - m), axis=-2, keepdims=True))


def _tc_scores_body(gf_ref, w1_ref, b1_ref, w2_ref, b2_ref, p_ref):
    gf = gf_ref[0]
    h = jnp.maximum(
        jnp.dot(gf, w1_ref[...], preferred_element_type=jnp.float32)
        + b1_ref[...], 0.0)
    scores = jnp.dot(h, w2_ref[...], preferred_element_type=jnp.float32) \
        + b2_ref[...]

    la = scores / TAU

    def sk(_, la):
        la = la - _lse_last(la)
        la = la - _lse_sub(la)
        return la

    la = lax.fori_loop(0, SINKHORN_ITERS, sk, la)
    p_ref[0] = jnp.exp(la)


def _store1(ref, idx, val, dtype):
    lanes = lax.iota(jnp.int32, 16)
    plsc.store_scatter(
        ref,
        [jnp.full((16,), idx, jnp.int32)],
        jnp.full((16,), val, dtype),
        mask=lanes == 0,
    )


def _sc_greedy_body(p_hbm, perm_hbm, pv, bound, colmask, supermax, permv):
    cid = lax.axis_index("c")
    sid = lax.axis_index("s")
    wid = sid * 2 + cid

    @pl.when(wid < B)
    def _():
        pltpu.sync_copy(p_hbm.at[wid], pv)
        lanes = lax.iota(jnp.int32, 16)

        def init_cm(k, carry):
            colmask[pl.ds(k * 16, 16)] = jnp.zeros((16,), jnp.float32)
            return carry

        lax.fori_loop(0, NCHUNK, init_cm, 0)

        def init_row(r, carry):
            m = pv[pl.ds(r * G, 16)]
            for k in range(1, NCHUNK):
                m = jnp.maximum(m, pv[pl.ds(r * G + k * 16, 16)])
            _store1(bound, r, jnp.max(m), jnp.float32)
            return carry

        lax.fori_loop(0, G, init_row, 0)

        def init_sm(j, carry):
            _store1(supermax, j, jnp.max(bound[pl.ds(j * 16, 16)]),
                    jnp.float32)
            return carry

        lax.fori_loop(0, 16, init_sm, 0)

        def cond(carry):
            n, it = carry
            return (n < G) & (it < G * G + G)

        def body(carry):
            n, it = carry
            sv = supermax[pl.ds(0, 16)]
            gm = jnp.max(sv)
            j = plsc.all_reduce_ffs(sv == gm)[0]
            bv = bound[pl.ds(j * 16, 16)]
            rloc = plsc.all_reduce_ffs(bv == gm)[0]
            r = j * 16 + rloc
            base = r * G
            best_v = pv[pl.ds(base, 16)] + colmask[pl.ds(0, 16)]
            best_i = lanes
            for k in range(1, NCHUNK):
                v = pv[pl.ds(base + k * 16, 16)] + colmask[pl.ds(k * 16, 16)]
                upd = v > best_v
                best_v = jnp.where(upd, v, best_v)
                best_i = jnp.where(upd, lanes + k * 16, best_i)
            m2 = jnp.max(best_v)
            cnd = jnp.min(jnp.where(best_v == m2, best_i, G))
            eq = m2 >= gm
            _store1(bound, r,
                    jnp.where(eq, jnp.float32(-1e9), m2), jnp.float32)
            _store1(colmask, jnp.where(eq, cnd, G), jnp.float32(-4.0),
                    jnp.float32)
            _store1(permv, jnp.where(eq, r, G), cnd, jnp.int32)
            _store1(supermax, j, jnp.max(bound[pl.ds(j * 16, 16)]),
                    jnp.float32)
            return (n + jnp.where(eq, 1, 0).astype(jnp.int32), it + 1)

        lax.while_loop(cond, body, (jnp.int32(0), jnp.int32(0)))
        pltpu.sync_copy(permv.at[pl.ds(0, G)], perm_hbm.at[wid])


_sc_greedy = functools.partial(
    pl.kernel,
    mesh=plsc.VectorSubcoreMesh(core_axis_name="c", subcore_axis_name="s"),
    out_type=jax.ShapeDtypeStruct((B, G), jnp.int32),
    compiler_params=pltpu.CompilerParams(needs_layout_passes=False),
    scratch_types=[
        pltpu.VMEM((G * G,), jnp.float32),
        pltpu.VMEM((G,), jnp.float32),
        pltpu.VMEM((G + 16,), jnp.float32),
        pltpu.VMEM((16,), jnp.float32),
        pltpu.VMEM((G + 16,), jnp.int32),
    ],
)(_sc_greedy_body)


def _tc_apply_body(perm_ref, cc_ref, gf_ref, rc_ref, rf_ref):
    pcol = perm_ref[0]
    col_ids = lax.broadcasted_iota(jnp.int32, (G, G), 1)
    M = (pcol == col_ids).astype(jnp.float32)
    contract = (((0,), (0,)), ((), ()))
    rf_ref[0] = lax.dot_general(M, gf_ref[0], contract,
                                preferred_element_type=jnp.float32,
                                precision=lax.Precision.HIGHEST)
    rc_ref[0] = lax.dot_general(M, cc_ref[0], contract,
                                preferred_element_type=jnp.float32,
                                precision=lax.Precision.HIGHEST)


def kernel(center_coords, group_features, W1, b1, W2, b2):
    b1r = b1.reshape(1, H)
    b2r = b2.reshape(1, G)
    P = pl.pallas_call(
        _tc_scores_body,
        grid=(B,),
        in_specs=[
            pl.BlockSpec((1, G, C), lambda b: (b, 0, 0)),
            pl.BlockSpec((C, H), lambda b: (0, 0)),
            pl.BlockSpec((1, H), lambda b: (0, 0)),
            pl.BlockSpec((H, G), lambda b: (0, 0)),
            pl.BlockSpec((1, G), lambda b: (0, 0)),
        ],
        out_specs=pl.BlockSpec((1, G, G), lambda b: (b, 0, 0)),
        out_shape=jax.ShapeDtypeStruct((B, G, G), jnp.float32),
    )(group_features, W1, b1r, W2, b2r)

    perm = _sc_greedy(P.reshape(B, G * G))

    rc, rf = pl.pallas_call(
        _tc_apply_body,
        grid=(B,),
        in_specs=[
            pl.BlockSpec((1, G, 1), lambda b: (b, 0, 0)),
            pl.BlockSpec((1, G, 3), lambda b: (b, 0, 0)),
            pl.BlockSpec((1, G, C), lambda b: (b, 0, 0)),
        ],
        out_specs=[
            pl.BlockSpec((1, G, 3), lambda b: (b, 0, 0)),
            pl.BlockSpec((1, G, C), lambda b: (b, 0, 0)),
        ],
        out_shape=[
            jax.ShapeDtypeStruct((B, G, 3), jnp.float32),
            jax.ShapeDtypeStruct((B, G, C), jnp.float32),
        ],
    )(perm.reshape(B, G, 1), center_coords, group_features)
    return (rc, rf, perm)


# speedup vs baseline: 1.1759x; 1.1220x over previous
import functools

import jax
import jax.numpy as jnp
from jax import lax
from jax.experimental import pallas as pl
from jax.experimental.pallas import tpu as pltpu
from jax.experimental.pallas import tpu_sc as plsc

B, G, C, H = 16, 256, 128, 256
TAU, SINKHORN_ITERS = 0.1, 10
NCHUNK = G // 16


def _lse_last(x):
    m = jnp.max(x, axis=-1, keepdims=True)
    return m + jnp.log(jnp.sum(jnp.exp(x - m), axis=-1, keepdims=True))


def _lse_sub(x):
    m = jnp.max(x, axis=-2, keepdims=True)
    return m + jnp.log(jnp.sum(jnp.exp(x - m), axis=-2, keepdims=True))


def _tc_scores_body(gf_ref, w1_ref, b1_ref, w2_ref, b2_ref, p_ref):
    gf = gf_ref[0]
    h = jnp.maximum(
        jnp.dot(gf, w1_ref[...], preferred_element_type=jnp.float32)
        + b1_ref[...], 0.0)
    scores = jnp.dot(h, w2_ref[...], preferred_element_type=jnp.float32) \
        + b2_ref[...]

    la = scores / TAU

    def sk(_, la):
        la = la - _lse_last(la)
        la = la - _lse_sub(la)
        return la

    la = lax.fori_loop(0, SINKHORN_ITERS, sk, la)
    p_ref[0] = jnp.exp(la)


def _store1(ref, idx, val, dtype):
    lanes = lax.iota(jnp.int32, 16)
    plsc.store_scatter(
        ref,
        [jnp.full((16,), idx, jnp.int32)],
        jnp.full((16,), val, dtype),
        mask=lanes == 0,
    )


def _sc_body(p_hbm, gf_hbm, cc_hbm, perm_hbm, rc_hbm, rf_hbm,
             pv, gf_v, cc_v, rc_v, bound, colmask, supermax, permv,
             idx_a, idx_b, sem_gf, sem_rf):
    cid = lax.axis_index("c")
    sid = lax.axis_index("s")
    wid = sid * 2 + cid

    @pl.when(wid < B)
    def _():
        gf_copy = pltpu.make_async_copy(gf_hbm.at[wid], gf_v, sem_gf)
        gf_copy.start()
        pltpu.sync_copy(p_hbm.at[wid], pv)
        pltpu.sync_copy(cc_hbm.at[wid], cc_v)
        lanes = lax.iota(jnp.int32, 16)

        def init_cm(k, carry):
            colmask[pl.ds(k * 16, 16)] = jnp.zeros((16,), jnp.float32)
            return carry

        lax.fori_loop(0, NCHUNK, init_cm, 0)

        def init_row(r, carry):
            m = pv[pl.ds(r * G, 16)]
            for k in range(1, NCHUNK):
                m = jnp.maximum(m, pv[pl.ds(r * G + k * 16, 16)])
            _store1(bound, r, jnp.max(m), jnp.float32)
            return carry

        lax.fori_loop(0, G, init_row, 0)

        def init_sm(j, carry):
            _store1(supermax, j, jnp.max(bound[pl.ds(j * 16, 16)]),
                    jnp.float32)
            return carry

        lax.fori_loop(0, 16, init_sm, 0)

        def cond(carry):
            n, it = carry
            return (n < G) & (it < G * G + G)

        def body(carry):
            n, it = carry
            sv = supermax[pl.ds(0, 16)]
            gm = jnp.max(sv)
            j = plsc.all_reduce_ffs(sv == gm)[0]
            bv = bound[pl.ds(j * 16, 16)]
            rloc = plsc.all_reduce_ffs(bv == gm)[0]
            r = j * 16 + rloc
            base = r * G
            best_v = pv[pl.ds(base, 16)] + colmask[pl.ds(0, 16)]
            best_i = lanes
            for k in range(1, NCHUNK):
                v = pv[pl.ds(base + k * 16, 16)] + colmask[pl.ds(k * 16, 16)]
                upd = v > best_v
                best_v = jnp.where(upd, v, best_v)
                best_i = jnp.where(upd, lanes + k * 16, best_i)
            m2 = jnp.max(best_v)
            cnd = jnp.min(jnp.where(best_v == m2, best_i, G))
            eq = m2 >= gm
            _store1(bound, r,
                    jnp.where(eq, jnp.float32(-1e9), m2), jnp.float32)
            _store1(colmask, jnp.where(eq, cnd, G), jnp.float32(-4.0),
                    jnp.float32)
            _store1(permv, jnp.where(eq, r, G), cnd, jnp.int32)
            _store1(supermax, j, jnp.max(bound[pl.ds(j * 16, 16)]),
                    jnp.float32)
            return (n + jnp.where(eq, 1, 0).astype(jnp.int32), it + 1)

        lax.while_loop(cond, body, (jnp.int32(0), jnp.int32(0)))
        pltpu.sync_copy(permv.at[pl.ds(0, G)], perm_hbm.at[wid])

        def rc_scatter(k, carry):
            pk = permv[pl.ds(k * 16, 16)]
            src = (k * 16 + lanes) * 3
            for d in range(3):
                vals = plsc.load_gather(cc_v, [src + d])
                plsc.store_scatter(rc_v, [pk * 3 + d], vals)
            return carry

        lax.fori_loop(0, NCHUNK, rc_scatter, 0)
        pltpu.sync_copy(rc_v, rc_hbm.at[wid])

        def mk_idx(k, carry):
            pk = permv[pl.ds(k * 16, 16)] + wid * G
            h = k - 8
            @pl.when(k < 8)
            def _():
                idx_a[pl.ds(k * 16, 16)] = pk
            @pl.when(k >= 8)
            def _():
                idx_b[pl.ds(h * 16, 16)] = pk
            return carry

        lax.fori_loop(0, NCHUNK, mk_idx, 0)
        gf_copy.wait()
        pltpu.make_async_copy(gf_v.at[pl.ds(0, G // 2)],
                              rf_hbm.at[idx_a], sem_rf).start()
        pltpu.make_async_copy(gf_v.at[pl.ds(G // 2, G // 2)],
                              rf_hbm.at[idx_b], sem_rf).start()
        pltpu.make_async_copy(gf_v.at[pl.ds(0, G // 2)],
                              rf_hbm.at[idx_a], sem_rf).wait()
        pltpu.make_async_copy(gf_v.at[pl.ds(G // 2, G // 2)],
                              rf_hbm.at[idx_b], sem_rf).wait()


_sc_greedy_apply = functools.partial(
    pl.kernel,
    mesh=plsc.VectorSubcoreMesh(core_axis_name="c", subcore_axis_name="s"),
    out_type=[
        jax.ShapeDtypeStruct((B, G), jnp.int32),
        jax.ShapeDtypeStruct((B, G * 3), jnp.float32),
        jax.ShapeDtypeStruct((B * G, C), jnp.float32),
    ],
    compiler_params=pltpu.CompilerParams(needs_layout_passes=False),
    scratch_types=[
        pltpu.VMEM((G * G,), jnp.float32),
        pltpu.VMEM((G, C), jnp.float32),
        pltpu.VMEM((G * 3,), jnp.float32),
        pltpu.VMEM((G * 3,), jnp.float32),
        pltpu.VMEM((G,), jnp.float32),
        pltpu.VMEM((G + 16,), jnp.float32),
        pltpu.VMEM((16,), jnp.float32),
        pltpu.VMEM((G + 16,), jnp.int32),
        pltpu.VMEM((G // 2,), jnp.int32),
        pltpu.VMEM((G // 2,), jnp.int32),
        pltpu.SemaphoreType.DMA,
        pltpu.SemaphoreType.DMA,
    ],
)(_sc_body)


def kernel(center_coords, group_features, W1, b1, W2, b2):
    b1r = b1.reshape(1, H)
    b2r = b2.reshape(1, G)
    P = pl.pallas_call(
        _tc_scores_body,
        grid=(B,),
        in_specs=[
            pl.BlockSpec((1, G, C), lambda b: (b, 0, 0)),
            pl.BlockSpec((C, H), lambda b: (0, 0)),
            pl.BlockSpec((1, H), lambda b: (0, 0)),
            pl.BlockSpec((H, G), lambda b: (0, 0)),
            pl.BlockSpec((1, G), lambda b: (0, 0)),
        ],
        out_specs=pl.BlockSpec((1, G, G), lambda b: (b, 0, 0)),
        out_shape=jax.ShapeDtypeStruct((B, G, G), jnp.float32),
    )(group_features, W1, b1r, W2, b2r)

    perm, rc_flat, rf_flat = _sc_greedy_apply(
        P.reshape(B, G * G), group_features, center_coords.reshape(B, G * 3))
    return (rc_flat.reshape(B, G, 3), rf_flat.reshape(B, G, C), perm)
